# baseline (speedup 1.0000x reference)
import functools

import jax
import jax.numpy as jnp
import numpy as np
from jax import lax
from jax.experimental import pallas as pl
from jax.experimental.pallas import tpu as pltpu
from jax.experimental.pallas import tpu_sc as plsc

ITEM_NUM = 100000
VDIM = 512
HID = 128
B = 1024
HL = 50

NC, NS = 2, 16
NW = NC * NS
CHUNK = 64
N_HIST_ROWS = 2 * B * HL
N_ROWS_RAW = N_HIST_ROWS + 3 * B
ROWS_PER_W = 3328
N_ROWS = ROWS_PER_W * NW
N_CHUNKS = ROWS_PER_W // CHUNK
SEGS = VDIM // 32
ROUND = np.int32(0x8000)
MASK_HI = np.int32(-65536)

BB = 64
NB = B // BB

_PERM = np.empty((VDIM,), np.int32)
for _p in range(VDIM):
    _s, _r = divmod(_p, 32)
    _PERM[_p] = 32 * _s + _r // 2 + (16 if _r % 2 else 0)


def _sc_gather_body(table_hbm, idx_hbm, out_hbm, idx0, idx1, rows0, rows1,
                    cvt0, cvt1, sem0, sem1):
    wid = lax.axis_index("s") * NC + lax.axis_index("c")
    base = wid * ROWS_PER_W
    idx_v = (idx0, idx1)
    rows_v = (rows0, rows1)
    cvt_v = (cvt0, cvt1)
    sems = (sem0, sem1)

    def start(j, b):
        off = base + j * CHUNK
        pltpu.sync_copy(idx_hbm.at[pl.ds(off, CHUNK)], idx_v[b])
        pltpu.async_copy(table_hbm.at[idx_v[b]], rows_v[b], sems[b])

    start(0, 0)
    start(1, 1)

    def outer(t, carry):
        j0 = t * 2
        for b in range(2):
            j = j0 + b
            pltpu.make_async_copy(table_hbm.at[idx_v[b]], rows_v[b],
                                  sems[b]).wait()

            @plsc.parallel_loop(0, CHUNK, unroll=2)
            def _cvt(row):
                for s in range(SEGS):
                    ua = rows_v[b][row, pl.ds(s * 32, 16)]
                    ub = rows_v[b][row, pl.ds(s * 32 + 16, 16)]
                    wa = lax.shift_right_logical(ua + ROUND, np.int32(16))
                    wb = (ub + ROUND) & MASK_HI
                    cvt_v[b][row, pl.ds(s * 16, 16)] = wa | wb

            pltpu.sync_copy(cvt_v[b],
                            out_hbm.at[pl.ds(base + j * CHUNK, CHUNK)])

            @pl.when(j + 2 < N_CHUNKS)
            def _():
                start(j + 2, b)
        return carry

    lax.fori_loop(0, N_CHUNKS // 2, outer, 0)


def _sc_gather(table, idx_all):
    mesh = plsc.VectorSubcoreMesh(core_axis_name="c", subcore_axis_name="s")
    f = functools.partial(
        pl.kernel,
        mesh=mesh,
        out_type=jax.ShapeDtypeStruct((N_ROWS, VDIM // 2), jnp.int32),
        scratch_types=[
            pltpu.VMEM((CHUNK,), jnp.int32),
            pltpu.VMEM((CHUNK,), jnp.int32),
            pltpu.VMEM((CHUNK, VDIM), jnp.int32),
            pltpu.VMEM((CHUNK, VDIM), jnp.int32),
            pltpu.VMEM((CHUNK, VDIM // 2), jnp.int32),
            pltpu.VMEM((CHUNK, VDIM // 2), jnp.int32),
            pltpu.SemaphoreType.DMA,
            pltpu.SemaphoreType.DMA,
        ],
    )(_sc_gather_body)
    return f(table, idx_all)


def _hist_body(g_ref, w_ref, b_ref, out_ref):
    x = g_ref[...]
    y = jnp.dot(x, w_ref[0], preferred_element_type=jnp.float32) + b_ref[0]
    s = jax.nn.sigmoid(y)
    m = jnp.mean(s.reshape(BB, HL, HID), axis=1)
    out_ref[0] = m


def _hist_means(g, w2, b2):
    return pl.pallas_call(
        _hist_body,
        grid=(2, NB),
        in_specs=[
            pl.BlockSpec((BB * HL, VDIM), lambda h, i: (h * NB + i, 0)),
            pl.BlockSpec((1, VDIM, HID), lambda h, i: (h, 0, 0)),
            pl.BlockSpec((1, 1, HID), lambda h, i: (h, 0, 0)),
        ],
        out_specs=pl.BlockSpec((1, BB, HID), lambda h, i: (h, i, 0)),
        out_shape=jax.ShapeDtypeStruct((2, B, HID), jnp.float32),
    )(g, w2, b2)


def _normed(x):
    n = jnp.sqrt(jnp.sum(x * x, axis=0, keepdims=True))
    return x / jnp.maximum(n, 1e-12)


def _cos_cols(a, b):
    num = jnp.sum(a * b, axis=-1, keepdims=True)
    den = jnp.sqrt(jnp.sum(a * a, axis=-1, keepdims=True)) * jnp.sqrt(
        jnp.sum(b * b, axis=-1, keepdims=True))
    return num / jnp.maximum(den, 1e-8)


def _final_body(gI_ref, gJ_ref, gK_ref, m_ref, wv_ref, bv_ref, ws_ref,
                bs_ref, w3_ref, b3_ref, out_ref):
    gI = gI_ref[...]
    gJ = gJ_ref[...]
    gK = gK_ref[...]

    def mlp(x, w_ref, b_ref):
        y = jnp.dot(x, w_ref[...],
                    preferred_element_type=jnp.float32) + b_ref[...]
        return jax.nn.sigmoid(y)

    I_lat = _normed(mlp(gI, wv_ref, bv_ref))
    J_lat = _normed(mlp(gJ, wv_ref, bv_ref))
    K_lat = _normed(mlp(gK, wv_ref, bv_ref))
    J_p = _normed(mlp(gJ, ws_ref, bs_ref))
    K_p = _normed(mlp(gK, ws_ref, bs_ref))
    J_c = _normed(mlp(gJ, w3_ref, b3_ref))
    K_c = _normed(mlp(gK, w3_ref, b3_ref))
    Mb = _normed(m_ref[0])
    Mt = _normed(m_ref[1])

    out_ref[:, 0:1] = _cos_cols(I_lat, J_lat)
    out_ref[:, 1:2] = _cos_cols(I_lat, K_lat)
    out_ref[:, 2:3] = _cos_cols(Mb, J_p)
    out_ref[:, 3:4] = _cos_cols(Mb, K_p)
    out_ref[:, 4:5] = _cos_cols(Mt, J_c)
    out_ref[:, 5:6] = _cos_cols(Mt, K_c)
    out_ref[:, 6:8] = jnp.zeros((B, 2), jnp.float32)


def _final(g, m, W_vis, b_vis, W_s, b_s, W_s3, b_s3):
    blk_I = N_HIST_ROWS // B
    return pl.pallas_call(
        _final_body,
        grid=(1,),
        in_specs=[
            pl.BlockSpec((B, VDIM), lambda i: (blk_I, 0)),
            pl.BlockSpec((B, VDIM), lambda i: (blk_I + 1, 0)),
            pl.BlockSpec((B, VDIM), lambda i: (blk_I + 2, 0)),
            pl.BlockSpec((2, B, HID), lambda i: (0, 0, 0)),
            pl.BlockSpec((VDIM, HID), lambda i: (0, 0)),
            pl.BlockSpec((1, HID), lambda i: (0, 0)),
            pl.BlockSpec((VDIM, HID), lambda i: (0, 0)),
            pl.BlockSpec((1, HID), lambda i: (0, 0)),
            pl.BlockSpec((VDIM, HID), lambda i: (0, 0)),
            pl.BlockSpec((1, HID), lambda i: (0, 0)),
        ],
        out_specs=pl.BlockSpec((B, 8), lambda i: (0, 0)),
        out_shape=jax.ShapeDtypeStruct((B, 8), jnp.float32),
    )(g, g, g, m, W_vis, b_vis.reshape(1, HID), W_s, b_s.reshape(1, HID),
      W_s3, b_s3.reshape(1, HID))


def kernel(Us, Is, Js, Ks, bhis, this, tbhis, train, visual_features,
           W_vis, b_vis, W_s, b_s, W_s3, b_s3):
    idx_all = jnp.concatenate([
        bhis.reshape(-1).astype(jnp.int32),
        this.reshape(-1).astype(jnp.int32),
        Is.astype(jnp.int32),
        Js.astype(jnp.int32),
        Ks.astype(jnp.int32),
        jnp.arange(N_ROWS - N_ROWS_RAW, dtype=jnp.int32),
    ])
    table_i32 = jax.lax.bitcast_convert_type(visual_features, jnp.int32)
    g_i32 = _sc_gather(table_i32, idx_all)
    g = jax.lax.bitcast_convert_type(g_i32, jnp.bfloat16).reshape(
        N_ROWS, VDIM)
    perm = jnp.asarray(_PERM)
    wv = W_vis[perm].astype(jnp.bfloat16)
    ws = W_s[perm].astype(jnp.bfloat16)
    w3 = W_s3[perm].astype(jnp.bfloat16)
    w2 = jnp.stack([ws, w3])
    b2 = jnp.stack([b_s, b_s3])
    m = _hist_means(g, w2, b2.reshape(2, 1, HID))
    out = _final(g, m, wv, b_vis, ws, b_s, w3, b_s3)
    return out.T[:6]


# speedup vs baseline: 4.8901x; 4.8901x over previous
import functools

import jax
import jax.numpy as jnp
from jax import lax
from jax.experimental import pallas as pl
from jax.experimental.pallas import tpu as pltpu
from jax.experimental.pallas import tpu_sc as plsc

ITEM_NUM = 100000
VDIM = 512
HID = 128
B = 1024
HL = 50

NC, NS = 2, 16
NW = NC * NS
N_HIST = B * HL
N_ROWS_RAW = 2 * N_HIST + 3 * B
N_ROWS = 106496
CHUNK = 104

BB = 64
NB = B // BB


@functools.lru_cache(maxsize=None)
def _make_gather(n_rows, chunk):
    rows_per_w = n_rows // NW
    n_chunks = rows_per_w // chunk
    assert rows_per_w % chunk == 0 and n_chunks % 2 == 0

    def body(table_hbm, idx_hbm, out_hbm, idx0, idx1, rows0, rows1,
             sem0, sem1):
        wid = lax.axis_index("s") * NC + lax.axis_index("c")
        base = wid * rows_per_w
        idx_v = (idx0, idx1)
        rows_v = (rows0, rows1)
        sems = (sem0, sem1)

        def start(j, b):
            off = base + j * chunk
            pltpu.sync_copy(idx_hbm.at[pl.ds(off, chunk)], idx_v[b])
            pltpu.async_copy(table_hbm.at[idx_v[b]], rows_v[b], sems[b])

        start(0, 0)
        start(1, 1)

        def outer(t, carry):
            j0 = t * 2
            for b in range(2):
                j = j0 + b
                pltpu.make_async_copy(table_hbm.at[idx_v[b]], rows_v[b],
                                      sems[b]).wait()
                pltpu.sync_copy(rows_v[b],
                                out_hbm.at[pl.ds(base + j * chunk, chunk)])

                @pl.when(j + 2 < n_chunks)
                def _():
                    start(j + 2, b)
            return carry

        lax.fori_loop(0, n_chunks // 2, outer, 0)

    mesh = plsc.VectorSubcoreMesh(core_axis_name="c", subcore_axis_name="s")
    return functools.partial(
        pl.kernel,
        mesh=mesh,
        out_type=jax.ShapeDtypeStruct((n_rows, VDIM), jnp.float32),
        scratch_types=[
            pltpu.VMEM((chunk,), jnp.int32),
            pltpu.VMEM((chunk,), jnp.int32),
            pltpu.VMEM((chunk, VDIM), jnp.float32),
            pltpu.VMEM((chunk, VDIM), jnp.float32),
            pltpu.SemaphoreType.DMA,
            pltpu.SemaphoreType.DMA,
        ],
    )(body)


def _normed(x):
    n = jnp.sqrt(jnp.sum(x * x, axis=0, keepdims=True))
    return x / jnp.maximum(n, 1e-12)


def _cos_cols(a, b):
    num = jnp.sum(a * b, axis=-1, keepdims=True)
    den = jnp.sqrt(jnp.sum(a * a, axis=-1, keepdims=True)) * jnp.sqrt(
        jnp.sum(b * b, axis=-1, keepdims=True))
    return num / jnp.maximum(den, 1e-8)


def _fused_body(g_ref, w2_ref, b2_ref, gI_ref, gJ_ref, gK_ref, wv_ref,
                bv_ref, ws_ref, bs_ref, w3_ref, b3_ref, out_ref,
                mb_acc, mt_acc):
    h = pl.program_id(0)
    i = pl.program_id(1)
    x = g_ref[...].astype(jnp.bfloat16)
    y = jnp.dot(x, w2_ref[0], preferred_element_type=jnp.float32) + b2_ref[0]
    s = jax.nn.sigmoid(y)
    m = jnp.mean(s.reshape(BB, HL, HID), axis=1)

    @pl.when(h == 0)
    def _():
        mb_acc[pl.ds(i * BB, BB), :] = m

    @pl.when(h == 1)
    def _():
        mt_acc[pl.ds(i * BB, BB), :] = m

    @pl.when((h == 1) & (i == NB - 1))
    def _():
        gI = gI_ref[...].astype(jnp.bfloat16)
        gJ = gJ_ref[...].astype(jnp.bfloat16)
        gK = gK_ref[...].astype(jnp.bfloat16)

        def mlp(xv, w_ref, b_ref):
            yv = jnp.dot(xv, w_ref[...],
                         preferred_element_type=jnp.float32) + b_ref[...]
            return jax.nn.sigmoid(yv)

        I_lat = _normed(mlp(gI, wv_ref, bv_ref))
        J_lat = _normed(mlp(gJ, wv_ref, bv_ref))
        K_lat = _normed(mlp(gK, wv_ref, bv_ref))
        J_p = _normed(mlp(gJ, ws_ref, bs_ref))
        K_p = _normed(mlp(gK, ws_ref, bs_ref))
        J_c = _normed(mlp(gJ, w3_ref, b3_ref))
        K_c = _normed(mlp(gK, w3_ref, b3_ref))
        Mb = _normed(mb_acc[...])
        Mt = _normed(mt_acc[...])

        out_ref[:, 0:1] = _cos_cols(I_lat, J_lat)
        out_ref[:, 1:2] = _cos_cols(I_lat, K_lat)
        out_ref[:, 2:3] = _cos_cols(Mb, J_p)
        out_ref[:, 3:4] = _cos_cols(Mb, K_p)
        out_ref[:, 4:5] = _cos_cols(Mt, J_c)
        out_ref[:, 5:6] = _cos_cols(Mt, K_c)
        out_ref[:, 6:8] = jnp.zeros((B, 2), jnp.float32)


def _fused(g, w2, b2, W_vis, b_vis, W_s, b_s, W_s3, b_s3):
    blk_I = 2 * N_HIST // B
    return pl.pallas_call(
        _fused_body,
        grid=(2, NB),
        in_specs=[
            pl.BlockSpec((BB * HL, VDIM), lambda h, i: (h * NB + i, 0)),
            pl.BlockSpec((1, VDIM, HID), lambda h, i: (h, 0, 0)),
            pl.BlockSpec((1, 1, HID), lambda h, i: (h, 0, 0)),
            pl.BlockSpec((B, VDIM), lambda h, i: (blk_I, 0)),
            pl.BlockSpec((B, VDIM), lambda h, i: (blk_I + 1, 0)),
            pl.BlockSpec((B, VDIM), lambda h, i: (blk_I + 2, 0)),
            pl.BlockSpec((VDIM, HID), lambda h, i: (0, 0)),
            pl.BlockSpec((1, HID), lambda h, i: (0, 0)),
            pl.BlockSpec((VDIM, HID), lambda h, i: (0, 0)),
            pl.BlockSpec((1, HID), lambda h, i: (0, 0)),
            pl.BlockSpec((VDIM, HID), lambda h, i: (0, 0)),
            pl.BlockSpec((1, HID), lambda h, i: (0, 0)),
        ],
        out_specs=pl.BlockSpec((B, 8), lambda h, i: (0, 0)),
        out_shape=jax.ShapeDtypeStruct((B, 8), jnp.float32),
        scratch_shapes=[pltpu.VMEM((B, HID), jnp.float32),
                        pltpu.VMEM((B, HID), jnp.float32)],
    )(g, w2, b2, g, g, g, W_vis, b_vis.reshape(1, HID), W_s,
      b_s.reshape(1, HID), W_s3, b_s3.reshape(1, HID))


def kernel(Us, Is, Js, Ks, bhis, this, tbhis, train, visual_features,
           W_vis, b_vis, W_s, b_s, W_s3, b_s3):
    idx_all = jnp.concatenate([
        bhis.reshape(-1).astype(jnp.int32),
        this.reshape(-1).astype(jnp.int32),
        Is.astype(jnp.int32),
        Js.astype(jnp.int32),
        Ks.astype(jnp.int32),
        jnp.arange(N_ROWS - N_ROWS_RAW, dtype=jnp.int32),
    ])
    g = _make_gather(N_ROWS, CHUNK)(visual_features, idx_all)
    w2 = jnp.stack([W_s, W_s3]).astype(jnp.bfloat16)
    b2 = jnp.stack([b_s, b_s3]).reshape(2, 1, HID)
    out = _fused(g, w2, b2, W_vis.astype(jnp.bfloat16), b_vis,
                 W_s.astype(jnp.bfloat16), b_s,
                 W_s3.astype(jnp.bfloat16), b_s3)
    return out.T[:6]
